# baseline (speedup 1.0000x reference)
import jax
import jax.numpy as jnp
from jax.experimental import pallas as pl
from jax.experimental.pallas import tpu as pltpu

DURATION = 100.0


def _body(x_ref, offp_ref, out_ref, acc_ref, occ_ref):
    i = pl.program_id(0)
    nsteps = pl.num_programs(0)
    nclass = out_ref.shape[1]

    blk = offp_ref[...]

    @pl.when(i == 0)
    def _init():
        acc_ref[...] = jnp.zeros_like(acc_ref)
        occ_ref[...] = jnp.zeros_like(occ_ref)

    x = ((DURATION - x_ref[...]) * (1.0 / DURATION)).astype(jnp.bfloat16)
    iota = jax.lax.broadcasted_iota(jnp.int32, (512, 16), 1)
    ones = jnp.ones((8, 512), jnp.bfloat16)
    acc = acc_ref[...]
    occ = occ_ref[...]
    for j in range(8):
        off = blk[:, 16 * j : 16 * (j + 1)]
        norms = jnp.sum(jnp.abs(off), axis=1, keepdims=True)
        prop = off / jnp.maximum(norms, 1e-12)
        maxv = jnp.max(prop, axis=1, keepdims=True)
        amax = jnp.min(jnp.where(prop == maxv, iota, 16), axis=1, keepdims=True)
        oh16 = iota == amax
        assoc = jnp.where(oh16, prop, 0.0)
        xj = x[:, 512 * j : 512 * (j + 1)]
        acc = acc + jnp.dot(
            xj, assoc.astype(jnp.bfloat16), preferred_element_type=jnp.float32
        )
        occ = occ + jnp.dot(
            ones, oh16.astype(jnp.bfloat16), preferred_element_type=jnp.float32
        )
    acc_ref[...] = acc
    occ_ref[...] = occ

    @pl.when(i == nsteps - 1)
    def _fini():
        occ = jnp.maximum(occ_ref[0:1, :nclass], 1.0)
        out_ref[...] = acc_ref[:, :nclass] / occ


def kernel(inputs, offsets):
    batch, nneuron = inputs.shape
    nclass = offsets.shape[1]
    blk_n = 4096
    grid = nneuron // blk_n

    offp = (
        jnp.pad(
            offsets.reshape(grid, 8, 512, nclass).transpose(2, 0, 1, 3),
            ((0, 0), (0, 0), (0, 0), (0, 16 - nclass)),
        )
    ).reshape(512, 8 * 16 * grid)

    return pl.pallas_call(
        _body,
        grid=(grid,),
        in_specs=[
            pl.BlockSpec((batch, blk_n), lambda i: (0, i)),
            pl.BlockSpec((512, 128), lambda i: (0, i)),
        ],
        out_specs=pl.BlockSpec((batch, nclass), lambda i: (0, 0)),
        out_shape=jax.ShapeDtypeStruct((batch, nclass), jnp.float32),
        scratch_shapes=[
            pltpu.VMEM((batch, 16), jnp.float32),
            pltpu.VMEM((8, 16), jnp.float32),
        ],
        compiler_params=pltpu.CompilerParams(
            dimension_semantics=("arbitrary",),
        ),
    )(inputs, offp)


# speedup vs baseline: 1.2892x; 1.2892x over previous
import jax
import jax.numpy as jnp
from jax.experimental import pallas as pl
from jax.experimental.pallas import tpu as pltpu

DURATION = 100.0


def _body(x_ref, offp_ref, out_ref, acc_ref, occ_ref):
    i = pl.program_id(0)
    nsteps = pl.num_programs(0)
    nclass = out_ref.shape[1]

    blk = offp_ref[...]
    off = jnp.concatenate(
        [blk[:, 16 * j : 16 * (j + 1)] for j in range(8)], axis=0
    )

    norms = jnp.sum(jnp.abs(off), axis=1, keepdims=True)
    prop = off / jnp.maximum(norms, 1e-12)
    maxv = jnp.max(prop, axis=1, keepdims=True)
    iota = jax.lax.broadcasted_iota(jnp.int32, prop.shape, 1)
    amax = jnp.min(jnp.where(prop == maxv, iota, 16), axis=1, keepdims=True)
    oh16 = iota == amax
    assoc = jnp.where(oh16, prop, 0.0)

    @pl.when(i == 0)
    def _init():
        acc_ref[...] = jnp.zeros_like(acc_ref)
        occ_ref[...] = jnp.zeros_like(occ_ref)

    x = ((DURATION - x_ref[...]) * (1.0 / DURATION)).astype(jnp.bfloat16)
    acc_ref[...] += jnp.dot(
        x, assoc.astype(jnp.bfloat16), preferred_element_type=jnp.float32
    )
    ones = jnp.ones((8, oh16.shape[0]), jnp.bfloat16)
    occ_ref[...] += jnp.dot(
        ones, oh16.astype(jnp.bfloat16), preferred_element_type=jnp.float32
    )

    @pl.when(i == nsteps - 1)
    def _fini():
        occ = jnp.maximum(occ_ref[0:1, :nclass], 1.0)
        out_ref[...] = acc_ref[:, :nclass] / occ


def kernel(inputs, offsets):
    batch, nneuron = inputs.shape
    nclass = offsets.shape[1]
    blk_n = 4096
    grid = nneuron // blk_n

    offp = (
        jnp.pad(
            offsets.reshape(grid, 8, 512, nclass).transpose(2, 0, 1, 3),
            ((0, 0), (0, 0), (0, 0), (0, 16 - nclass)),
        )
    ).reshape(512, 8 * 16 * grid)

    return pl.pallas_call(
        _body,
        grid=(grid,),
        in_specs=[
            pl.BlockSpec((batch, blk_n), lambda i: (0, i)),
            pl.BlockSpec((512, 128), lambda i: (0, i)),
        ],
        out_specs=pl.BlockSpec((batch, nclass), lambda i: (0, 0)),
        out_shape=jax.ShapeDtypeStruct((batch, nclass), jnp.float32),
        scratch_shapes=[
            pltpu.VMEM((batch, 16), jnp.float32),
            pltpu.VMEM((8, 16), jnp.float32),
        ],
        compiler_params=pltpu.CompilerParams(
            dimension_semantics=("arbitrary",),
        ),
    )(inputs, offp)
